# Initial kernel scaffold; baseline (speedup 1.0000x reference)
import jax
import jax.numpy as jnp
from jax.experimental import pallas as pl


def kernel(out, labels):
    raise NotImplementedError("write your pallas kernel here")


# speedup vs baseline: 1.7830x; 1.7830x over previous
import jax
import jax.numpy as jnp
from jax.experimental import pallas as pl
from jax.experimental.pallas import tpu as pltpu

_NB = 16
_NA = 3
_GS = 52
_N = _NA * _GS * _GS
_NPAD = 8192
_M = 300
_IGNORE_THRES = 0.5
_EPS = 1e-16
_BIG = 2 ** 30


def _loss_kernel(pred_ref, lab_ref, out_ref):
    b = pl.program_id(0)
    bf = b.astype(jnp.float32)

    raw = pred_ref[0]
    r0 = raw[0:1, :]
    r1 = raw[1:2, :]
    r2 = raw[2:3, :]
    r3 = raw[3:4, :]
    r4 = raw[4:5, :]

    lane = jax.lax.broadcasted_iota(jnp.int32, (1, _NPAD), 1)
    row = jax.lax.broadcasted_iota(jnp.int32, (_M, 1), 0)
    col_valid = lane < _N

    cell = jax.lax.rem(lane, jnp.int32(_GS * _GS))
    gxv = jax.lax.rem(cell, jnp.int32(_GS)).astype(jnp.float32)
    gyv = (cell // _GS).astype(jnp.float32)

    inv_gs = jnp.float32(1.0 / _GS)
    xs = (jnp.tanh(r0) + 0.5 + gxv) * inv_gs
    ys = (jnp.tanh(r1) + 0.5 + gyv) * inv_gs
    ws = jnp.exp(-jnp.square(r2))
    hs = jnp.exp(-jnp.square(r3))
    conf = jax.nn.sigmoid(r4)
    lw_p = -jnp.square(r2)
    lh_p = -jnp.square(r3)

    px1 = xs - ws * 0.5
    px2 = xs + ws * 0.5
    py1 = ys - hs * 0.5
    py2 = ys + hs * 0.5
    area_p = (px2 - px1) * (py2 - py1)

    gbid = lab_ref[:, 0:1]
    gxc = lab_ref[:, 1:2]
    gyc = lab_ref[:, 2:3]
    gw = lab_ref[:, 3:4]
    gh = lab_ref[:, 4:5]
    valid = gbid == bf
    any_valid = jnp.max(jnp.where(valid, 1, 0)) > 0

    gx1 = gxc - gw * 0.5
    gx2 = gxc + gw * 0.5
    gy1 = gyc - gh * 0.5
    gy2 = gyc + gh * 0.5
    area_g = (gx2 - gx1) * (gy2 - gy1)

    iw = jnp.maximum(jnp.minimum(px2, gx2) - jnp.maximum(px1, gx1), 0.0)
    ih = jnp.maximum(jnp.minimum(py2, gy2) - jnp.maximum(py1, gy1), 0.0)
    inter = iw * ih
    union = area_p + area_g - inter + _EPS
    ious = inter / union
    cw = jnp.maximum(px2, gx2) - jnp.minimum(px1, gx1)
    ch = jnp.maximum(py2, gy2) - jnp.minimum(py1, gy1)
    c_area = cw * ch + _EPS
    gous = ious - (c_area - union) / c_area

    neg_inf = jnp.float32(-jnp.inf)
    gous_c = jnp.where(col_valid, gous, neg_inf)

    def cond_fn(state):
        sel_i, _, _, _ = state
        return jnp.min(sel_i) == 0

    def body_fn(state):
        sel_i, msk_i, pi, pidx = state
        selected = sel_i > 0
        mask_p = msk_i > 0
        gous_m = jnp.where(mask_p, neg_inf, gous_c)
        rmax = jnp.max(gous_m, axis=1, keepdims=True)
        is_m = gous_m == rmax
        pd_j = jnp.min(jnp.where(is_m, lane, _BIG), axis=1, keepdims=True)
        onehot = lane == pd_j
        val_i = jnp.sum(jnp.where(onehot, ious, 0.0), axis=1, keepdims=True)
        elig = ~selected
        key = jnp.where(elig, rmax, neg_inf)
        bv = jnp.max(jnp.where(onehot, key, neg_inf), axis=0, keepdims=True)
        bv_at = jnp.max(jnp.where(onehot, bv, neg_inf), axis=1, keepdims=True)
        cand = elig & (rmax == bv_at)
        jkey = jnp.where(cand, row, -1)
        bj = jnp.max(jnp.where(onehot, jkey, -1), axis=0, keepdims=True)
        bj_at = jnp.max(jnp.where(onehot, bj, -1), axis=1, keepdims=True)
        winner = cand & (row == bj_at)
        win_oh = onehot & winner
        haswin_i = jnp.max(
            jnp.where(win_oh, 1, 0), axis=0, keepdims=True)
        haswin = haswin_i > 0
        pi_upd = jnp.sum(jnp.where(win_oh, val_i, 0.0), axis=0, keepdims=True)
        pidx_upd = jnp.sum(jnp.where(win_oh, row, 0), axis=0, keepdims=True)
        pi = jnp.where(haswin, pi_upd, pi)
        pidx = jnp.where(haswin, pidx_upd, pidx)
        msk_i = msk_i | haswin_i
        sel_i = sel_i | jnp.where(winner, 1, 0)
        return sel_i, msk_i, pi, pidx

    init = (
        jnp.where(valid, 0, 1),
        jnp.zeros((1, _NPAD), dtype=jnp.int32),
        jnp.zeros((1, _NPAD), dtype=jnp.float32),
        jnp.zeros((1, _NPAD), dtype=jnp.int32),
    )
    sel_f, msk_f, pi, pidx = jax.lax.while_loop(cond_fn, body_fn, init)
    mask_p = msk_f > 0

    gvalid = jnp.where(valid, gous_c, neg_inf)
    colmax = jnp.max(gvalid, axis=0, keepdims=True)
    gt_idx2 = jnp.min(
        jnp.where(gvalid == colmax, row, _BIG), axis=0, keepdims=True)
    onehot2 = row == gt_idx2
    fb_i = jnp.sum(jnp.where(onehot2, ious, 0.0), axis=0, keepdims=True)
    pi_f = jnp.where(mask_p, pi, jnp.where(any_valid, fb_i, 0.0))

    match_oh = (row == pidx) & mask_p
    mo = match_oh.astype(jnp.float32)
    s_x = jnp.sum(mo * jnp.square(xs - gxc))
    s_y = jnp.sum(mo * jnp.square(ys - gyc))
    s_w = jnp.sum(mo * jnp.square(lw_p - jnp.log(gw)))
    s_h = jnp.sum(mo * jnp.square(lh_p - jnp.log(gh)))
    s_gou = jnp.sum(mo * gous)
    s_iou = jnp.sum(mo * ious)

    t = mask_p.astype(jnp.float32)
    thr = pi_f > _IGNORE_THRES
    ig = ((~thr) | mask_p) & col_valid
    igf = ig.astype(jnp.float32)
    pc = jnp.clip(conf, 1e-7, 1.0 - 1e-7)
    bce = -(t * jnp.log(pc) + (1.0 - t) * jnp.log(1.0 - pc))
    s_bce = jnp.sum(bce * igf)
    s_cnt = jnp.sum(igf)
    s_inter = jnp.sum(conf * t * igf)
    s_p2 = jnp.sum(jnp.square(conf) * igf)
    s_t2 = jnp.sum(t * t * igf)

    sums = (s_x, s_y, s_w, s_h, s_gou, s_iou,
            s_bce, s_cnt, s_inter, s_p2, s_t2)
    vec = jnp.zeros((1, 128), dtype=jnp.float32)
    for k, s in enumerate(sums):
        vec = vec + jnp.where(lane[:, :128] == k, s, 0.0)
    out_ref[...] = jnp.broadcast_to(vec[None], (1, 8, 128))


def kernel(out, labels):
    nb = out.shape[0]
    pred = out.reshape(nb, _NA, 5, _GS * _GS).transpose(0, 2, 1, 3)
    pred = pred.reshape(nb, 5, _N)
    pred = jnp.pad(pred, ((0, 0), (0, 0), (0, _NPAD - _N)))

    partials = pl.pallas_call(
        _loss_kernel,
        grid=(nb,),
        in_specs=[
            pl.BlockSpec((1, 5, _NPAD), lambda b: (b, 0, 0)),
            pl.BlockSpec((_M, 5), lambda b: (0, 0)),
        ],
        out_specs=pl.BlockSpec((1, 8, 128), lambda b: (b, 0, 0)),
        out_shape=jax.ShapeDtypeStruct((nb, 8, 128), jnp.float32),
        compiler_params=pltpu.CompilerParams(
            dimension_semantics=("arbitrary",),
            vmem_limit_bytes=100 * 1024 * 1024,
        ),
    )(pred, labels)

    s = jnp.sum(partials[:, 0, :], axis=0)
    m = jnp.float32(_M)
    loss_xy = (s[0] + s[1]) / m
    loss_wh = (s[2] + s[3]) / m
    loss_gou = 1.0 - s[4] / m
    loss_iou = 1.0 - s[5] / m
    loss_conf_bce = s[6] / s[7]
    dice = (2.0 * s[8] + 1.0) / (s[9] + s[10] + 1.0)
    dice = jnp.where(jnp.isnan(dice), 1.0, dice)
    loss_conf = (1.0 - dice) + loss_conf_bce
    total = loss_conf + 2.0 * loss_gou + loss_wh + loss_xy
    return (total, loss_xy, loss_wh, loss_conf, loss_iou, loss_gou)


# speedup vs baseline: 2.0824x; 1.1680x over previous
import jax
import jax.numpy as jnp
from jax.experimental import pallas as pl
from jax.experimental.pallas import tpu as pltpu

_NB = 16
_NA = 3
_GS = 52
_N = _NA * _GS * _GS
_NPAD = 8192
_M = 300
_IGNORE_THRES = 0.5
_EPS = 1e-16
_BIG = 2 ** 30


def _loss_kernel(pred_ref, lab_ref, out_ref):
    b = pl.program_id(0)
    bf = b.astype(jnp.float32)

    raw = pred_ref[0]
    r0 = raw[0:1, :]
    r1 = raw[1:2, :]
    r2 = raw[2:3, :]
    r3 = raw[3:4, :]
    r4 = raw[4:5, :]

    lane = jax.lax.broadcasted_iota(jnp.int32, (1, _NPAD), 1)
    row = jax.lax.broadcasted_iota(jnp.int32, (_M, 1), 0)
    col_valid = lane < _N

    cell = jax.lax.rem(lane, jnp.int32(_GS * _GS))
    gxv = jax.lax.rem(cell, jnp.int32(_GS)).astype(jnp.float32)
    gyv = (cell // _GS).astype(jnp.float32)

    inv_gs = jnp.float32(1.0 / _GS)
    xs = (jnp.tanh(r0) + 0.5 + gxv) * inv_gs
    ys = (jnp.tanh(r1) + 0.5 + gyv) * inv_gs
    ws = jnp.exp(-jnp.square(r2))
    hs = jnp.exp(-jnp.square(r3))
    conf = jax.nn.sigmoid(r4)
    lw_p = -jnp.square(r2)
    lh_p = -jnp.square(r3)

    px1 = xs - ws * 0.5
    px2 = xs + ws * 0.5
    py1 = ys - hs * 0.5
    py2 = ys + hs * 0.5
    area_p = (px2 - px1) * (py2 - py1)

    gbid = lab_ref[:, 0:1]
    gxc = lab_ref[:, 1:2]
    gyc = lab_ref[:, 2:3]
    gw = lab_ref[:, 3:4]
    gh = lab_ref[:, 4:5]
    valid = gbid == bf
    any_valid = jnp.max(jnp.where(valid, 1, 0)) > 0

    gx1 = gxc - gw * 0.5
    gx2 = gxc + gw * 0.5
    gy1 = gyc - gh * 0.5
    gy2 = gyc + gh * 0.5
    area_g = (gx2 - gx1) * (gy2 - gy1)

    iw = jnp.maximum(jnp.minimum(px2, gx2) - jnp.maximum(px1, gx1), 0.0)
    ih = jnp.maximum(jnp.minimum(py2, gy2) - jnp.maximum(py1, gy1), 0.0)
    inter = iw * ih
    union = area_p + area_g - inter + _EPS
    ious = inter / union
    cw = jnp.maximum(px2, gx2) - jnp.minimum(px1, gx1)
    ch = jnp.maximum(py2, gy2) - jnp.minimum(py1, gy1)
    c_area = cw * ch + _EPS
    gous = ious - (c_area - union) / c_area

    neg_inf = jnp.float32(-jnp.inf)
    gous_c = jnp.where(col_valid, gous, neg_inf)

    def cond_fn(state):
        sel_i, _, _ = state
        return jnp.min(sel_i) == 0

    def body_fn(state):
        sel_i, msk_i, pidx = state
        mask_p = msk_i > 0
        gous_m = jnp.where(mask_p, neg_inf, gous_c)
        rmax = jnp.max(gous_m, axis=1, keepdims=True)
        is_m = gous_m == rmax
        pd_j = jnp.min(jnp.where(is_m, lane, _BIG), axis=1, keepdims=True)
        onehot = lane == pd_j
        elig = sel_i == 0
        key = jnp.where(elig, rmax, neg_inf)
        bv = jnp.max(jnp.where(onehot, key, neg_inf), axis=0, keepdims=True)
        bv_at = jnp.max(jnp.where(onehot, bv, neg_inf), axis=1, keepdims=True)
        cand = elig & (rmax == bv_at)
        jkey = jnp.where(cand, row, -1)
        bj = jnp.max(jnp.where(onehot, jkey, -1), axis=0, keepdims=True)
        bj_at = jnp.max(jnp.where(onehot, bj, -1), axis=1, keepdims=True)
        winner = cand & (row == bj_at)
        win_oh = onehot & winner
        haswin_i = jnp.max(
            jnp.where(win_oh, 1, 0), axis=0, keepdims=True)
        pidx_upd = jnp.sum(jnp.where(win_oh, row, 0), axis=0, keepdims=True)
        pidx = jnp.where(haswin_i > 0, pidx_upd, pidx)
        msk_i = msk_i | haswin_i
        sel_i = sel_i | jnp.where(winner, 1, 0)
        return sel_i, msk_i, pidx

    init = (
        jnp.where(valid, 0, 1),
        jnp.zeros((1, _NPAD), dtype=jnp.int32),
        jnp.zeros((1, _NPAD), dtype=jnp.int32),
    )
    sel_f, msk_f, pidx = jax.lax.while_loop(cond_fn, body_fn, init)
    mask_p = msk_f > 0

    gvalid = jnp.where(valid, gous_c, neg_inf)
    colmax = jnp.max(gvalid, axis=0, keepdims=True)
    gt_idx2 = jnp.min(
        jnp.where(gvalid == colmax, row, _BIG), axis=0, keepdims=True)
    onehot2 = row == gt_idx2
    fb_i = jnp.sum(jnp.where(onehot2, ious, 0.0), axis=0, keepdims=True)

    match_oh = (row == pidx) & mask_p
    mo = match_oh.astype(jnp.float32)
    pi_m = jnp.sum(mo * ious, axis=0, keepdims=True)
    pi_f = jnp.where(mask_p, pi_m, jnp.where(any_valid, fb_i, 0.0))
    s_x = jnp.sum(mo * jnp.square(xs - gxc))
    s_y = jnp.sum(mo * jnp.square(ys - gyc))
    s_w = jnp.sum(mo * jnp.square(lw_p - jnp.log(gw)))
    s_h = jnp.sum(mo * jnp.square(lh_p - jnp.log(gh)))
    s_gou = jnp.sum(mo * gous)
    s_iou = jnp.sum(pi_m)

    t = mask_p.astype(jnp.float32)
    thr = pi_f > _IGNORE_THRES
    ig = ((~thr) | mask_p) & col_valid
    igf = ig.astype(jnp.float32)
    pc = jnp.clip(conf, 1e-7, 1.0 - 1e-7)
    bce = -(t * jnp.log(pc) + (1.0 - t) * jnp.log(1.0 - pc))
    s_bce = jnp.sum(bce * igf)
    s_cnt = jnp.sum(igf)
    s_inter = jnp.sum(conf * t * igf)
    s_p2 = jnp.sum(jnp.square(conf) * igf)
    s_t2 = jnp.sum(t * t * igf)

    sums = (s_x, s_y, s_w, s_h, s_gou, s_iou,
            s_bce, s_cnt, s_inter, s_p2, s_t2)
    vec = jnp.zeros((1, 128), dtype=jnp.float32)
    for k, s in enumerate(sums):
        vec = vec + jnp.where(lane[:, :128] == k, s, 0.0)
    out_ref[...] = jnp.broadcast_to(vec[None], (1, 8, 128))


def kernel(out, labels):
    nb = out.shape[0]
    pred = out.reshape(nb, _NA, 5, _GS * _GS).transpose(0, 2, 1, 3)
    pred = pred.reshape(nb, 5, _N)
    pred = jnp.pad(pred, ((0, 0), (0, 0), (0, _NPAD - _N)))

    partials = pl.pallas_call(
        _loss_kernel,
        grid=(nb,),
        in_specs=[
            pl.BlockSpec((1, 5, _NPAD), lambda b: (b, 0, 0)),
            pl.BlockSpec((_M, 5), lambda b: (0, 0)),
        ],
        out_specs=pl.BlockSpec((1, 8, 128), lambda b: (b, 0, 0)),
        out_shape=jax.ShapeDtypeStruct((nb, 8, 128), jnp.float32),
        compiler_params=pltpu.CompilerParams(
            dimension_semantics=("parallel",),
            vmem_limit_bytes=100 * 1024 * 1024,
        ),
    )(pred, labels)

    s = jnp.sum(partials[:, 0, :], axis=0)
    m = jnp.float32(_M)
    loss_xy = (s[0] + s[1]) / m
    loss_wh = (s[2] + s[3]) / m
    loss_gou = 1.0 - s[4] / m
    loss_iou = 1.0 - s[5] / m
    loss_conf_bce = s[6] / s[7]
    dice = (2.0 * s[8] + 1.0) / (s[9] + s[10] + 1.0)
    dice = jnp.where(jnp.isnan(dice), 1.0, dice)
    loss_conf = (1.0 - dice) + loss_conf_bce
    total = loss_conf + 2.0 * loss_gou + loss_wh + loss_xy
    return (total, loss_xy, loss_wh, loss_conf, loss_iou, loss_gou)
